# Initial kernel scaffold; baseline (speedup 1.0000x reference)
import functools

import jax
import jax.numpy as jnp
from jax import lax
from jax.experimental import pallas as pl
from jax.experimental.pallas import tpu as pltpu
from jax.experimental.pallas import tpu_sc as plsc

NN = 10000
EE = 320000
DD = 128
TW = DD + 16

NC = 2
NS = 16
NW = NC * NS
EPT = EE // NW
CH = 80
NCH = EPT // CH
RPT = NN // NS

_f32 = jnp.float32


def _gather_body(table_hbm, src_hbm, dst_hbm, gs_hbm, gd_hbm,
                 idx_s, idx_d, buf_s, buf_d, sem_s, sem_d):
    c = lax.axis_index("c")
    s = lax.axis_index("s")
    wid = s * NC + c
    base = wid * EPT

    def step(i, carry):
        off = base + i * CH
        pltpu.sync_copy(src_hbm.at[pl.ds(off, CH)], idx_s)
        pltpu.sync_copy(dst_hbm.at[pl.ds(off, CH)], idx_d)
        cp_s = pltpu.async_copy(table_hbm.at[idx_s], buf_s, sem_s)
        cp_d = pltpu.async_copy(table_hbm.at[idx_d], buf_d, sem_d)
        cp_s.wait()
        cp_d.wait()
        pltpu.sync_copy(buf_s, gs_hbm.at[pl.ds(off, CH)])
        pltpu.sync_copy(buf_d, gd_hbm.at[pl.ds(off, CH)])
        return carry

    lax.fori_loop(0, NCH, step, 0)


_gather = functools.partial(
    pl.kernel,
    out_type=(jax.ShapeDtypeStruct((EE, TW), _f32),
              jax.ShapeDtypeStruct((EE, TW), _f32)),
    mesh=plsc.VectorSubcoreMesh(core_axis_name="c", subcore_axis_name="s",
                                num_cores=NC, num_subcores=NS),
    scratch_types=[
        pltpu.VMEM((CH,), jnp.int32),
        pltpu.VMEM((CH,), jnp.int32),
        pltpu.VMEM((CH, TW), _f32),
        pltpu.VMEM((CH, TW), _f32),
        pltpu.SemaphoreType.DMA,
        pltpu.SemaphoreType.DMA,
    ],
)(_gather_body)


def _scatter_body(ef_hbm, aux_hbm, dst_hbm, z128_hbm, z16_hbm,
                  pef_hbm, paux_hbm,
                  acc128, acc16, idxb, efb, auxb, vbuf128, vbuf16):
    c = lax.axis_index("c")
    s = lax.axis_index("s")
    wid = s * NC + c
    base = wid * EPT
    r0 = s * RPT

    pltpu.sync_copy(z128_hbm.at[pl.ds(r0, RPT)], vbuf128)
    pltpu.sync_copy(vbuf128, acc128.at[pl.ds(r0, RPT)])
    pltpu.sync_copy(z16_hbm.at[pl.ds(r0, RPT)], vbuf16)
    pltpu.sync_copy(vbuf16, acc16.at[pl.ds(r0, RPT)])
    plsc.subcore_barrier()

    def step(i, carry):
        off = base + i * CH
        pltpu.sync_copy(dst_hbm.at[pl.ds(off, CH)], idxb)
        pltpu.sync_copy(ef_hbm.at[pl.ds(off, CH)], efb)
        pltpu.sync_copy(aux_hbm.at[pl.ds(off, CH)], auxb)
        pltpu.sync_copy(efb, acc128.at[idxb], add=True)
        pltpu.sync_copy(auxb, acc16.at[idxb], add=True)
        return carry

    lax.fori_loop(0, NCH, step, 0)
    plsc.subcore_barrier()

    pltpu.sync_copy(acc128.at[pl.ds(r0, RPT)], vbuf128)
    pltpu.sync_copy(vbuf128, pef_hbm.at[c].at[pl.ds(r0, RPT)])
    pltpu.sync_copy(acc16.at[pl.ds(r0, RPT)], vbuf16)
    pltpu.sync_copy(vbuf16, paux_hbm.at[c].at[pl.ds(r0, RPT)])


_scatter = functools.partial(
    pl.kernel,
    out_type=(jax.ShapeDtypeStruct((NC, NN, DD), _f32),
              jax.ShapeDtypeStruct((NC, NN, 16), _f32)),
    mesh=plsc.VectorSubcoreMesh(core_axis_name="c", subcore_axis_name="s",
                                num_cores=NC, num_subcores=NS),
    scratch_types=[
        pltpu.VMEM_SHARED((NN, DD), _f32),
        pltpu.VMEM_SHARED((NN, 16), _f32),
        pltpu.VMEM((CH,), jnp.int32),
        pltpu.VMEM((CH, DD), _f32),
        pltpu.VMEM((CH, 16), _f32),
        pltpu.VMEM((RPT, DD), _f32),
        pltpu.VMEM((RPT, 16), _f32),
    ],
)(_scatter_body)


BE = 512


def _edge_body(gs_ref, gd_ref, w1a_ref, w1b_ref, w1c_ref, be1_ref,
               w2_ref, be2_ref, wc1_ref, bc1_ref, wc2_ref,
               ef_ref, aux_ref):
    gs = gs_ref[...]
    gd = gd_ref[...]
    fs = gs[:, :DD]
    fd = gd[:, :DD]
    diff = gd[:, DD:TW] - gs[:, DD:TW]
    radial = jnp.sum(diff * diff, axis=1, keepdims=True)
    h1 = (jnp.dot(fs, w1a_ref[...], preferred_element_type=_f32)
          + jnp.dot(fd, w1b_ref[...], preferred_element_type=_f32)
          + radial * w1c_ref[...]
          + be1_ref[...])
    h1 = jnp.maximum(h1, 0.0)
    ef = jnp.maximum(jnp.dot(h1, w2_ref[...], preferred_element_type=_f32)
                     + be2_ref[...], 0.0)
    c1 = jnp.maximum(jnp.dot(ef, wc1_ref[...], preferred_element_type=_f32)
                     + bc1_ref[...], 0.0)
    w = jnp.sum(c1 * wc2_ref[...], axis=1, keepdims=True)
    aux = diff * w
    lanes = lax.broadcasted_iota(jnp.int32, (BE, 16), 1)
    aux = jnp.where(lanes == 3, 1.0, aux)
    ef_ref[...] = ef
    aux_ref[...] = aux


def _edge_mlp(gs, gd, w1a, w1b, w1c, be1, w2, be2, wc1, bc1, wc2):
    nblk = EE // BE
    full128 = pl.BlockSpec((DD, DD), lambda i: (0, 0))
    row128 = pl.BlockSpec((1, DD), lambda i: (0, 0))
    return pl.pallas_call(
        _edge_body,
        grid=(nblk,),
        in_specs=[
            pl.BlockSpec((BE, TW), lambda i: (i, 0)),
            pl.BlockSpec((BE, TW), lambda i: (i, 0)),
            full128, full128, row128, row128,
            full128, row128, full128, row128, row128,
        ],
        out_specs=[
            pl.BlockSpec((BE, DD), lambda i: (i, 0)),
            pl.BlockSpec((BE, 16), lambda i: (i, 0)),
        ],
        out_shape=[
            jax.ShapeDtypeStruct((EE, DD), _f32),
            jax.ShapeDtypeStruct((EE, 16), _f32),
        ],
    )(gs, gd, w1a, w1b, w1c, be1, w2, be2, wc1, bc1, wc2)


BN = 1000


def _node_body(nf_ref, c16_ref, pef_ref, paux_ref,
               wn1a_ref, wn1b_ref, bn1_ref, wn2_ref, bn2_ref,
               nfo_ref, co_ref):
    agg = pef_ref[0] + pef_ref[1]
    aggaux = paux_ref[0] + paux_ref[1]
    cnt = jnp.maximum(aggaux[:, 3:4], 1.0)
    delta = aggaux / cnt
    lanes = lax.broadcasted_iota(jnp.int32, (BN, 16), 1)
    delta = jnp.where(lanes < 3, delta, 0.0)
    co_ref[...] = c16_ref[...] + delta
    nf = nf_ref[...]
    h = jnp.maximum(jnp.dot(nf, wn1a_ref[...], preferred_element_type=_f32)
                    + jnp.dot(agg, wn1b_ref[...], preferred_element_type=_f32)
                    + bn1_ref[...], 0.0)
    nfo_ref[...] = (nf + jnp.dot(h, wn2_ref[...], preferred_element_type=_f32)
                    + bn2_ref[...])


def _node_mlp(nf, c16, pef, paux, wn1a, wn1b, bn1, wn2, bn2):
    nblk = NN // BN
    full128 = pl.BlockSpec((DD, DD), lambda i: (0, 0))
    row128 = pl.BlockSpec((1, DD), lambda i: (0, 0))
    return pl.pallas_call(
        _node_body,
        grid=(nblk,),
        in_specs=[
            pl.BlockSpec((BN, DD), lambda i: (i, 0)),
            pl.BlockSpec((BN, 16), lambda i: (i, 0)),
            pl.BlockSpec((NC, BN, DD), lambda i: (0, i, 0)),
            pl.BlockSpec((NC, BN, 16), lambda i: (0, i, 0)),
            full128, full128, row128, full128, row128,
        ],
        out_specs=[
            pl.BlockSpec((BN, DD), lambda i: (i, 0)),
            pl.BlockSpec((BN, 16), lambda i: (i, 0)),
        ],
        out_shape=[
            jax.ShapeDtypeStruct((NN, DD), _f32),
            jax.ShapeDtypeStruct((NN, 16), _f32),
        ],
    )(nf, c16, pef, paux, wn1a, wn1b, bn1, wn2, bn2)


def kernel(node_feat, coord, edge_list, We1, be1, We2, be2,
           Wn1, bn1, Wn2, bn2, Wc1, bc1, Wc2):
    src = edge_list[:, 0]
    dst = edge_list[:, 1]
    coord16 = jnp.pad(coord, ((0, 0), (0, 13)))
    table = jnp.concatenate([node_feat, coord16], axis=1)

    gs, gd = _gather(table, src, dst)

    w1a = We1[:DD]
    w1b = We1[DD:2 * DD]
    w1c = We1[2 * DD:2 * DD + 1]
    ef, aux = _edge_mlp(gs, gd, w1a, w1b, w1c, be1.reshape(1, DD),
                        We2, be2.reshape(1, DD), Wc1, bc1.reshape(1, DD),
                        Wc2.reshape(1, DD))

    z128 = jnp.zeros((NN, DD), _f32)
    z16 = jnp.zeros((NN, 16), _f32)
    pef, paux = _scatter(ef, aux, dst, z128, z16)

    nfo, co16 = _node_mlp(node_feat, coord16, pef, paux,
                          Wn1[:DD], Wn1[DD:2 * DD], bn1.reshape(1, DD),
                          Wn2, bn2.reshape(1, DD))
    return (nfo, co16[:, :3])


# speedup vs baseline: 3.3328x; 3.3328x over previous
import functools

import jax
import jax.numpy as jnp
from jax import lax
from jax.experimental import pallas as pl
from jax.experimental.pallas import tpu as pltpu
from jax.experimental.pallas import tpu_sc as plsc

NN = 10000
EE = 320000
DD = 128

NC = 2
NS = 16
NW = NC * NS
EPT = EE // NW
CH = 80
NCH = EPT // CH
G = CH // 16
NP = 10240
RPT = NP // NS

_f32 = jnp.float32
_mesh = plsc.VectorSubcoreMesh(core_axis_name="c", subcore_axis_name="s",
                               num_cores=NC, num_subcores=NS)
_sc_params = pltpu.CompilerParams(needs_layout_passes=False)


def _gather_body(nf_hbm, cx_hbm, cy_hbm, cz_hbm, src_hbm, dst_hbm,
                 gs_hbm, gd_hbm, dx_hbm, dy_hbm, dz_hbm,
                 idx_s, idx_d, buf_s, buf_d, cxv, cyv, czv, dxb, dyb, dzb,
                 sem_s, sem_d):
    c = lax.axis_index("c")
    s = lax.axis_index("s")
    wid = s * NC + c
    base = wid * EPT

    pltpu.sync_copy(cx_hbm, cxv)
    pltpu.sync_copy(cy_hbm, cyv)
    pltpu.sync_copy(cz_hbm, czv)

    def step(i, carry):
        off = base + i * CH
        pltpu.sync_copy(src_hbm.at[pl.ds(off, CH)], idx_s)
        pltpu.sync_copy(dst_hbm.at[pl.ds(off, CH)], idx_d)
        cp_s = pltpu.async_copy(nf_hbm.at[idx_s], buf_s, sem_s)
        cp_d = pltpu.async_copy(nf_hbm.at[idx_d], buf_d, sem_d)
        for g in range(G):
            sv = idx_s[pl.ds(g * 16, 16)]
            dv = idx_d[pl.ds(g * 16, 16)]
            dxb[pl.ds(g * 16, 16)] = (plsc.load_gather(cxv, [dv])
                                      - plsc.load_gather(cxv, [sv]))
            dyb[pl.ds(g * 16, 16)] = (plsc.load_gather(cyv, [dv])
                                      - plsc.load_gather(cyv, [sv]))
            dzb[pl.ds(g * 16, 16)] = (plsc.load_gather(czv, [dv])
                                      - plsc.load_gather(czv, [sv]))
        cp_s.wait()
        cp_d.wait()
        pltpu.sync_copy(buf_s, gs_hbm.at[pl.ds(off, CH)])
        pltpu.sync_copy(buf_d, gd_hbm.at[pl.ds(off, CH)])
        pltpu.sync_copy(dxb, dx_hbm.at[pl.ds(off, CH)])
        pltpu.sync_copy(dyb, dy_hbm.at[pl.ds(off, CH)])
        pltpu.sync_copy(dzb, dz_hbm.at[pl.ds(off, CH)])
        return carry

    lax.fori_loop(0, NCH, step, 0)


_gather = functools.partial(
    pl.kernel,
    out_type=(jax.ShapeDtypeStruct((EE, DD), _f32),
              jax.ShapeDtypeStruct((EE, DD), _f32),
              jax.ShapeDtypeStruct((EE,), _f32),
              jax.ShapeDtypeStruct((EE,), _f32),
              jax.ShapeDtypeStruct((EE,), _f32)),
    mesh=_mesh,
    compiler_params=_sc_params,
    scratch_types=[
        pltpu.VMEM((CH,), jnp.int32),
        pltpu.VMEM((CH,), jnp.int32),
        pltpu.VMEM((CH, DD), _f32),
        pltpu.VMEM((CH, DD), _f32),
        pltpu.VMEM((NN,), _f32),
        pltpu.VMEM((NN,), _f32),
        pltpu.VMEM((NN,), _f32),
        pltpu.VMEM((CH,), _f32),
        pltpu.VMEM((CH,), _f32),
        pltpu.VMEM((CH,), _f32),
        pltpu.SemaphoreType.DMA,
        pltpu.SemaphoreType.DMA,
    ],
)(_gather_body)


def _scat_ef_body(ef_hbm, dst_hbm, z128_hbm, pef_hbm, acc128, idxb, efb):
    c = lax.axis_index("c")
    s = lax.axis_index("s")
    wid = s * NC + c
    base = wid * EPT
    r0 = s * RPT

    pltpu.sync_copy(z128_hbm.at[pl.ds(r0, RPT)], acc128.at[pl.ds(r0, RPT)])
    plsc.subcore_barrier()

    def step(i, carry):
        off = base + i * CH
        pltpu.sync_copy(dst_hbm.at[pl.ds(off, CH)], idxb)
        pltpu.sync_copy(ef_hbm.at[pl.ds(off, CH)], efb)
        pltpu.sync_copy(efb, acc128.at[idxb], add=True)
        return carry

    lax.fori_loop(0, NCH, step, 0)
    plsc.subcore_barrier()

    pltpu.sync_copy(acc128.at[pl.ds(r0, RPT)], pef_hbm.at[c].at[pl.ds(r0, RPT)])


_scat_ef = functools.partial(
    pl.kernel,
    out_type=jax.ShapeDtypeStruct((NC, NP, DD), _f32),
    mesh=_mesh,
    compiler_params=_sc_params,
    scratch_types=[
        pltpu.VMEM_SHARED((NP, DD), _f32),
        pltpu.VMEM((CH,), jnp.int32),
        pltpu.VMEM((CH, DD), _f32),
    ],
)(_scat_ef_body)


def _scat_aux_body(dst_hbm, wx_hbm, wy_hbm, wz_hbm, z4_hbm, paux_hbm,
                   acc4, idxb, wxb, wyb, wzb):
    c = lax.axis_index("c")
    s = lax.axis_index("s")
    wid = s * NC + c
    base = wid * EPT

    pltpu.sync_copy(z4_hbm, acc4)
    ones16 = jnp.ones((16,), _f32)

    def step(i, carry):
        off = base + i * CH
        pltpu.sync_copy(dst_hbm.at[pl.ds(off, CH)], idxb)
        pltpu.sync_copy(wx_hbm.at[pl.ds(off, CH)], wxb)
        pltpu.sync_copy(wy_hbm.at[pl.ds(off, CH)], wyb)
        pltpu.sync_copy(wz_hbm.at[pl.ds(off, CH)], wzb)
        for g in range(G):
            dv = idxb[pl.ds(g * 16, 16)]
            plsc.addupdate_scatter(acc4, [dv], wxb[pl.ds(g * 16, 16)])
            plsc.addupdate_scatter(acc4, [dv + NP], wyb[pl.ds(g * 16, 16)])
            plsc.addupdate_scatter(acc4, [dv + 2 * NP], wzb[pl.ds(g * 16, 16)])
            plsc.addupdate_scatter(acc4, [dv + 3 * NP], ones16)
        return carry

    lax.fori_loop(0, NCH, step, 0)
    pltpu.sync_copy(acc4, paux_hbm.at[wid])


_scat_aux = functools.partial(
    pl.kernel,
    out_type=jax.ShapeDtypeStruct((NW, 4 * NP), _f32),
    mesh=_mesh,
    compiler_params=_sc_params,
    scratch_types=[
        pltpu.VMEM((4 * NP,), _f32),
        pltpu.VMEM((CH,), jnp.int32),
        pltpu.VMEM((CH,), _f32),
        pltpu.VMEM((CH,), _f32),
        pltpu.VMEM((CH,), _f32),
    ],
)(_scat_aux_body)


BE = 512


def _edge_body(gs_ref, gd_ref, dx_ref, dy_ref, dz_ref,
               w1a_ref, w1b_ref, w1c_ref, be1_ref,
               w2_ref, be2_ref, wc1_ref, bc1_ref, wc2_ref,
               ef_ref, wx_ref, wy_ref, wz_ref):
    fs = gs_ref[...]
    fd = gd_ref[...]
    dx = dx_ref[...]
    dy = dy_ref[...]
    dz = dz_ref[...]
    radial = (dx * dx + dy * dy + dz * dz).reshape(BE, 1)
    h1 = (jnp.dot(fs, w1a_ref[...], preferred_element_type=_f32)
          + jnp.dot(fd, w1b_ref[...], preferred_element_type=_f32)
          + radial * w1c_ref[...]
          + be1_ref[...])
    h1 = jnp.maximum(h1, 0.0)
    ef = jnp.maximum(jnp.dot(h1, w2_ref[...], preferred_element_type=_f32)
                     + be2_ref[...], 0.0)
    c1 = jnp.maximum(jnp.dot(ef, wc1_ref[...], preferred_element_type=_f32)
                     + bc1_ref[...], 0.0)
    w = jnp.sum(c1 * wc2_ref[...], axis=1)
    ef_ref[...] = ef
    wx_ref[...] = dx * w
    wy_ref[...] = dy * w
    wz_ref[...] = dz * w


def _edge_mlp(gs, gd, dx, dy, dz, w1a, w1b, w1c, be1, w2, be2, wc1, bc1, wc2):
    nblk = EE // BE
    full128 = pl.BlockSpec((DD, DD), lambda i: (0, 0))
    row128 = pl.BlockSpec((1, DD), lambda i: (0, 0))
    vec = pl.BlockSpec((BE,), lambda i: (i,))
    return pl.pallas_call(
        _edge_body,
        grid=(nblk,),
        in_specs=[
            pl.BlockSpec((BE, DD), lambda i: (i, 0)),
            pl.BlockSpec((BE, DD), lambda i: (i, 0)),
            vec, vec, vec,
            full128, full128, row128, row128,
            full128, row128, full128, row128, row128,
        ],
        out_specs=[
            pl.BlockSpec((BE, DD), lambda i: (i, 0)),
            vec, vec, vec,
        ],
        out_shape=[
            jax.ShapeDtypeStruct((EE, DD), _f32),
            jax.ShapeDtypeStruct((EE,), _f32),
            jax.ShapeDtypeStruct((EE,), _f32),
            jax.ShapeDtypeStruct((EE,), _f32),
        ],
    )(gs, gd, dx, dy, dz, w1a, w1b, w1c, be1, w2, be2, wc1, bc1, wc2)


def _node_body(nf_ref, ct_ref, pef_ref, paux_ref,
               wn1a_ref, wn1b_ref, bn1_ref, wn2_ref, bn2_ref,
               nfo_ref, cot_ref):
    agg = pef_ref[0] + pef_ref[1]
    pa = paux_ref[...]
    wx = jnp.sum(pa[:, 0:NP], axis=0, keepdims=True)
    wy = jnp.sum(pa[:, NP:2 * NP], axis=0, keepdims=True)
    wz = jnp.sum(pa[:, 2 * NP:3 * NP], axis=0, keepdims=True)
    cnt = jnp.maximum(jnp.sum(pa[:, 3 * NP:4 * NP], axis=0, keepdims=True), 1.0)
    inv = 1.0 / cnt
    delta = jnp.concatenate(
        [wx * inv, wy * inv, wz * inv, jnp.zeros((5, NP), _f32)], axis=0)
    cot_ref[...] = ct_ref[...] + delta
    nf = nf_ref[...]
    h = jnp.maximum(jnp.dot(nf, wn1a_ref[...], preferred_element_type=_f32)
                    + jnp.dot(agg, wn1b_ref[...], preferred_element_type=_f32)
                    + bn1_ref[...], 0.0)
    nfo_ref[...] = (nf + jnp.dot(h, wn2_ref[...], preferred_element_type=_f32)
                    + bn2_ref[...])


def _node_mlp(nf, ct8, pef, paux, wn1a, wn1b, bn1, wn2, bn2):
    return pl.pallas_call(
        _node_body,
        out_shape=[
            jax.ShapeDtypeStruct((NP, DD), _f32),
            jax.ShapeDtypeStruct((8, NP), _f32),
        ],
    )(nf, ct8, pef, paux, wn1a, wn1b, bn1, wn2, bn2)


def kernel(node_feat, coord, edge_list, We1, be1, We2, be2,
           Wn1, bn1, Wn2, bn2, Wc1, bc1, Wc2):
    src = edge_list[:, 0]
    dst = edge_list[:, 1]
    cx = coord[:, 0]
    cy = coord[:, 1]
    cz = coord[:, 2]
    ct8 = jnp.pad(coord.T, ((0, 5), (0, NP - NN)))
    nfp = jnp.pad(node_feat, ((0, NP - NN), (0, 0)))

    gs, gd, dx, dy, dz = _gather(node_feat, cx, cy, cz, src, dst)

    w1a = We1[:DD]
    w1b = We1[DD:2 * DD]
    w1c = We1[2 * DD:2 * DD + 1]
    ef, wx, wy, wz = _edge_mlp(gs, gd, dx, dy, dz, w1a, w1b, w1c,
                               be1.reshape(1, DD), We2, be2.reshape(1, DD),
                               Wc1, bc1.reshape(1, DD), Wc2.reshape(1, DD))

    z128 = jnp.zeros((NP, DD), _f32)
    z4 = jnp.zeros((4 * NP,), _f32)
    pef = _scat_ef(ef, dst, z128)
    paux = _scat_aux(dst, wx, wy, wz, z4)

    nfo, cot = _node_mlp(nfp, ct8, pef, paux,
                         Wn1[:DD], Wn1[DD:2 * DD], bn1.reshape(1, DD),
                         Wn2, bn2.reshape(1, DD))
    return (nfo[:NN], cot[:3, :NN].T)


# speedup vs baseline: 3.7088x; 1.1128x over previous
import functools

import jax
import jax.numpy as jnp
from jax import lax
from jax.experimental import pallas as pl
from jax.experimental.pallas import tpu as pltpu
from jax.experimental.pallas import tpu_sc as plsc

NN = 10000
EE = 320000
DD = 128

NC = 2
NS = 16
NW = NC * NS
EPT = EE // NW
CH = 80
NCH = EPT // CH
G = CH // 16
CHA = 400
NCHA = EPT // CHA
GA = CHA // 16
NP = 10240
RPT = NP // NS

_f32 = jnp.float32
_mesh = plsc.VectorSubcoreMesh(core_axis_name="c", subcore_axis_name="s",
                               num_cores=NC, num_subcores=NS)
_sc_params = pltpu.CompilerParams(needs_layout_passes=False)


def _gather_body(nf_hbm, cx_hbm, cy_hbm, cz_hbm, src_hbm, dst_hbm,
                 gs_hbm, gd_hbm, dx_hbm, dy_hbm, dz_hbm,
                 idx_s, idx_d, buf_s, buf_d, cxv, cyv, czv, dxb, dyb, dzb,
                 sem_s, sem_d):
    c = lax.axis_index("c")
    s = lax.axis_index("s")
    wid = s * NC + c
    base = wid * EPT

    pltpu.sync_copy(cx_hbm, cxv)
    pltpu.sync_copy(cy_hbm, cyv)
    pltpu.sync_copy(cz_hbm, czv)

    def step(i, carry):
        off = base + i * CH
        pltpu.sync_copy(src_hbm.at[pl.ds(off, CH)], idx_s)
        pltpu.sync_copy(dst_hbm.at[pl.ds(off, CH)], idx_d)
        cp_s = pltpu.async_copy(nf_hbm.at[idx_s], buf_s, sem_s)
        cp_d = pltpu.async_copy(nf_hbm.at[idx_d], buf_d, sem_d)
        for g in range(G):
            sv = idx_s[pl.ds(g * 16, 16)]
            dv = idx_d[pl.ds(g * 16, 16)]
            dxb[pl.ds(g * 16, 16)] = (plsc.load_gather(cxv, [dv])
                                      - plsc.load_gather(cxv, [sv]))
            dyb[pl.ds(g * 16, 16)] = (plsc.load_gather(cyv, [dv])
                                      - plsc.load_gather(cyv, [sv]))
            dzb[pl.ds(g * 16, 16)] = (plsc.load_gather(czv, [dv])
                                      - plsc.load_gather(czv, [sv]))
        cp_s.wait()
        cp_d.wait()
        pltpu.sync_copy(buf_s, gs_hbm.at[pl.ds(off, CH)])
        pltpu.sync_copy(buf_d, gd_hbm.at[pl.ds(off, CH)])
        pltpu.sync_copy(dxb, dx_hbm.at[pl.ds(off, CH)])
        pltpu.sync_copy(dyb, dy_hbm.at[pl.ds(off, CH)])
        pltpu.sync_copy(dzb, dz_hbm.at[pl.ds(off, CH)])
        return carry

    lax.fori_loop(0, NCH, step, 0)


_gather = functools.partial(
    pl.kernel,
    out_type=(jax.ShapeDtypeStruct((EE, DD), _f32),
              jax.ShapeDtypeStruct((EE, DD), _f32),
              jax.ShapeDtypeStruct((EE,), _f32),
              jax.ShapeDtypeStruct((EE,), _f32),
              jax.ShapeDtypeStruct((EE,), _f32)),
    mesh=_mesh,
    compiler_params=_sc_params,
    scratch_types=[
        pltpu.VMEM((CH,), jnp.int32),
        pltpu.VMEM((CH,), jnp.int32),
        pltpu.VMEM((CH, DD), _f32),
        pltpu.VMEM((CH, DD), _f32),
        pltpu.VMEM((NN,), _f32),
        pltpu.VMEM((NN,), _f32),
        pltpu.VMEM((NN,), _f32),
        pltpu.VMEM((CH,), _f32),
        pltpu.VMEM((CH,), _f32),
        pltpu.VMEM((CH,), _f32),
        pltpu.SemaphoreType.DMA,
        pltpu.SemaphoreType.DMA,
    ],
)(_gather_body)


def _scat_ef_body(ef_hbm, dst_hbm, z128_hbm, pef_hbm, acc128, idxb, efb):
    c = lax.axis_index("c")
    s = lax.axis_index("s")
    wid = s * NC + c
    base = wid * EPT
    r0 = s * RPT

    pltpu.sync_copy(z128_hbm.at[pl.ds(r0, RPT)], acc128.at[pl.ds(r0, RPT)])
    plsc.subcore_barrier()

    def step(i, carry):
        off = base + i * CH
        pltpu.sync_copy(dst_hbm.at[pl.ds(off, CH)], idxb)
        pltpu.sync_copy(ef_hbm.at[pl.ds(off, CH)], efb)
        pltpu.sync_copy(efb, acc128.at[idxb], add=True)
        return carry

    lax.fori_loop(0, NCH, step, 0)
    plsc.subcore_barrier()

    pltpu.sync_copy(acc128.at[pl.ds(r0, RPT)], pef_hbm.at[c].at[pl.ds(r0, RPT)])


_scat_ef = functools.partial(
    pl.kernel,
    out_type=jax.ShapeDtypeStruct((NC, NP, DD), _f32),
    mesh=_mesh,
    compiler_params=_sc_params,
    scratch_types=[
        pltpu.VMEM_SHARED((NP, DD), _f32),
        pltpu.VMEM((CH,), jnp.int32),
        pltpu.VMEM((CH, DD), _f32),
    ],
)(_scat_ef_body)


def _scat_aux_body(dst_hbm, wx_hbm, wy_hbm, wz_hbm, z4_hbm, paux_hbm,
                   acc4, idxb, wxb, wyb, wzb):
    c = lax.axis_index("c")
    s = lax.axis_index("s")
    wid = s * NC + c
    base = wid * EPT

    pltpu.sync_copy(z4_hbm, acc4)
    ones16 = jnp.ones((16,), _f32)

    def step(i, carry):
        off = base + i * CHA
        pltpu.sync_copy(dst_hbm.at[pl.ds(off, CHA)], idxb)
        pltpu.sync_copy(wx_hbm.at[pl.ds(off, CHA)], wxb)
        pltpu.sync_copy(wy_hbm.at[pl.ds(off, CHA)], wyb)
        pltpu.sync_copy(wz_hbm.at[pl.ds(off, CHA)], wzb)
        for g in range(GA):
            dv = idxb[pl.ds(g * 16, 16)]
            plsc.addupdate_scatter(acc4, [dv], wxb[pl.ds(g * 16, 16)])
            plsc.addupdate_scatter(acc4, [dv + NP], wyb[pl.ds(g * 16, 16)])
            plsc.addupdate_scatter(acc4, [dv + 2 * NP], wzb[pl.ds(g * 16, 16)])
            plsc.addupdate_scatter(acc4, [dv + 3 * NP], ones16)
        return carry

    lax.fori_loop(0, NCHA, step, 0)
    pltpu.sync_copy(acc4, paux_hbm.at[wid])


_scat_aux = functools.partial(
    pl.kernel,
    out_type=jax.ShapeDtypeStruct((NW, 4 * NP), _f32),
    mesh=_mesh,
    compiler_params=_sc_params,
    scratch_types=[
        pltpu.VMEM((4 * NP,), _f32),
        pltpu.VMEM((CHA,), jnp.int32),
        pltpu.VMEM((CHA,), _f32),
        pltpu.VMEM((CHA,), _f32),
        pltpu.VMEM((CHA,), _f32),
    ],
)(_scat_aux_body)


BE = 512


def _edge_body(gs_ref, gd_ref, dx_ref, dy_ref, dz_ref,
               w1a_ref, w1b_ref, w1c_ref, be1_ref,
               w2_ref, be2_ref, wc1_ref, bc1_ref, wc2_ref,
               ef_ref, wx_ref, wy_ref, wz_ref):
    fs = gs_ref[...]
    fd = gd_ref[...]
    dx = dx_ref[...]
    dy = dy_ref[...]
    dz = dz_ref[...]
    radial = (dx * dx + dy * dy + dz * dz).reshape(BE, 1)
    h1 = (jnp.dot(fs, w1a_ref[...], preferred_element_type=_f32)
          + jnp.dot(fd, w1b_ref[...], preferred_element_type=_f32)
          + radial * w1c_ref[...]
          + be1_ref[...])
    h1 = jnp.maximum(h1, 0.0)
    ef = jnp.maximum(jnp.dot(h1, w2_ref[...], preferred_element_type=_f32)
                     + be2_ref[...], 0.0)
    c1 = jnp.maximum(jnp.dot(ef, wc1_ref[...], preferred_element_type=_f32)
                     + bc1_ref[...], 0.0)
    w = jnp.sum(c1 * wc2_ref[...], axis=1)
    ef_ref[...] = ef
    wx_ref[...] = dx * w
    wy_ref[...] = dy * w
    wz_ref[...] = dz * w


def _edge_mlp(gs, gd, dx, dy, dz, w1a, w1b, w1c, be1, w2, be2, wc1, bc1, wc2):
    nblk = EE // BE
    full128 = pl.BlockSpec((DD, DD), lambda i: (0, 0))
    row128 = pl.BlockSpec((1, DD), lambda i: (0, 0))
    vec = pl.BlockSpec((BE,), lambda i: (i,))
    return pl.pallas_call(
        _edge_body,
        grid=(nblk,),
        in_specs=[
            pl.BlockSpec((BE, DD), lambda i: (i, 0)),
            pl.BlockSpec((BE, DD), lambda i: (i, 0)),
            vec, vec, vec,
            full128, full128, row128, row128,
            full128, row128, full128, row128, row128,
        ],
        out_specs=[
            pl.BlockSpec((BE, DD), lambda i: (i, 0)),
            vec, vec, vec,
        ],
        out_shape=[
            jax.ShapeDtypeStruct((EE, DD), _f32),
            jax.ShapeDtypeStruct((EE,), _f32),
            jax.ShapeDtypeStruct((EE,), _f32),
            jax.ShapeDtypeStruct((EE,), _f32),
        ],
    )(gs, gd, dx, dy, dz, w1a, w1b, w1c, be1, w2, be2, wc1, bc1, wc2)


def _node_body(nf_ref, ct_ref, pef_ref, paux_ref,
               wn1a_ref, wn1b_ref, bn1_ref, wn2_ref, bn2_ref,
               nfo_ref, cot_ref):
    agg = pef_ref[0] + pef_ref[1]
    pa = paux_ref[...]
    wx = jnp.sum(pa[:, 0:NP], axis=0, keepdims=True)
    wy = jnp.sum(pa[:, NP:2 * NP], axis=0, keepdims=True)
    wz = jnp.sum(pa[:, 2 * NP:3 * NP], axis=0, keepdims=True)
    cnt = jnp.maximum(jnp.sum(pa[:, 3 * NP:4 * NP], axis=0, keepdims=True), 1.0)
    inv = 1.0 / cnt
    delta = jnp.concatenate(
        [wx * inv, wy * inv, wz * inv, jnp.zeros((5, NP), _f32)], axis=0)
    cot_ref[...] = ct_ref[...] + delta
    nf = nf_ref[...]
    h = jnp.maximum(jnp.dot(nf, wn1a_ref[...], preferred_element_type=_f32)
                    + jnp.dot(agg, wn1b_ref[...], preferred_element_type=_f32)
                    + bn1_ref[...], 0.0)
    nfo_ref[...] = (nf + jnp.dot(h, wn2_ref[...], preferred_element_type=_f32)
                    + bn2_ref[...])


def _node_mlp(nf, ct8, pef, paux, wn1a, wn1b, bn1, wn2, bn2):
    return pl.pallas_call(
        _node_body,
        out_shape=[
            jax.ShapeDtypeStruct((NP, DD), _f32),
            jax.ShapeDtypeStruct((8, NP), _f32),
        ],
    )(nf, ct8, pef, paux, wn1a, wn1b, bn1, wn2, bn2)


def kernel(node_feat, coord, edge_list, We1, be1, We2, be2,
           Wn1, bn1, Wn2, bn2, Wc1, bc1, Wc2):
    src = edge_list[:, 0]
    dst = edge_list[:, 1]
    cx = coord[:, 0]
    cy = coord[:, 1]
    cz = coord[:, 2]
    ct8 = jnp.pad(coord.T, ((0, 5), (0, NP - NN)))
    nfp = jnp.pad(node_feat, ((0, NP - NN), (0, 0)))

    gs, gd, dx, dy, dz = _gather(node_feat, cx, cy, cz, src, dst)

    w1a = We1[:DD]
    w1b = We1[DD:2 * DD]
    w1c = We1[2 * DD:2 * DD + 1]
    ef, wx, wy, wz = _edge_mlp(gs, gd, dx, dy, dz, w1a, w1b, w1c,
                               be1.reshape(1, DD), We2, be2.reshape(1, DD),
                               Wc1, bc1.reshape(1, DD), Wc2.reshape(1, DD))

    z128 = jnp.zeros((NP, DD), _f32)
    z4 = jnp.zeros((4 * NP,), _f32)
    pef = _scat_ef(ef, dst, z128)
    paux = _scat_aux(dst, wx, wy, wz, z4)

    nfo, cot = _node_mlp(nfp, ct8, pef, paux,
                         Wn1[:DD], Wn1[DD:2 * DD], bn1.reshape(1, DD),
                         Wn2, bn2.reshape(1, DD))
    return (nfo[:NN], cot[:3, :NN].T)


# speedup vs baseline: 3.8239x; 1.0310x over previous
import functools

import jax
import jax.numpy as jnp
from jax import lax
from jax.experimental import pallas as pl
from jax.experimental.pallas import tpu as pltpu
from jax.experimental.pallas import tpu_sc as plsc

NN = 10000
EE = 320000
DD = 128

NC = 2
NS = 16
NW = NC * NS
EPT = EE // NW
CH = 80
NCH = EPT // CH
G = CH // 16
CHA = 400
NCHA = EPT // CHA
GA = CHA // 16
NP = 10240
RPT = NP // NS

_f32 = jnp.float32
_mesh = plsc.VectorSubcoreMesh(core_axis_name="c", subcore_axis_name="s",
                               num_cores=NC, num_subcores=NS)
_sc_params = pltpu.CompilerParams(needs_layout_passes=False)


def _gather_body(p1_hbm, p2_hbm, cx_hbm, cy_hbm, cz_hbm, src_hbm, dst_hbm,
                 hp_hbm, dx_hbm, dy_hbm, dz_hbm,
                 idx_s, idx_d, buf_s, buf_d, cxv, cyv, czv, dxb, dyb, dzb,
                 sem_s, sem_d):
    c = lax.axis_index("c")
    s = lax.axis_index("s")
    wid = s * NC + c
    base = wid * EPT

    pltpu.sync_copy(cx_hbm, cxv)
    pltpu.sync_copy(cy_hbm, cyv)
    pltpu.sync_copy(cz_hbm, czv)

    def step(i, carry):
        off = base + i * CH
        pltpu.sync_copy(src_hbm.at[pl.ds(off, CH)], idx_s)
        pltpu.sync_copy(dst_hbm.at[pl.ds(off, CH)], idx_d)
        cp_s = pltpu.async_copy(p1_hbm.at[idx_s], buf_s, sem_s)
        cp_d = pltpu.async_copy(p2_hbm.at[idx_d], buf_d, sem_d)
        for g in range(G):
            sv = idx_s[pl.ds(g * 16, 16)]
            dv = idx_d[pl.ds(g * 16, 16)]
            dxb[pl.ds(g * 16, 16)] = (plsc.load_gather(cxv, [dv])
                                      - plsc.load_gather(cxv, [sv]))
            dyb[pl.ds(g * 16, 16)] = (plsc.load_gather(cyv, [dv])
                                      - plsc.load_gather(cyv, [sv]))
            dzb[pl.ds(g * 16, 16)] = (plsc.load_gather(czv, [dv])
                                      - plsc.load_gather(czv, [sv]))
        cp_s.wait()
        cp_d.wait()

        def row(j, carry2):
            for g in range(DD // 16):
                sl = pl.ds(g * 16, 16)
                buf_s[j, sl] = buf_s[j, sl] + buf_d[j, sl]
            return carry2

        lax.fori_loop(0, CH, row, 0)
        pltpu.sync_copy(buf_s, hp_hbm.at[pl.ds(off, CH)])
        pltpu.sync_copy(dxb, dx_hbm.at[pl.ds(off, CH)])
        pltpu.sync_copy(dyb, dy_hbm.at[pl.ds(off, CH)])
        pltpu.sync_copy(dzb, dz_hbm.at[pl.ds(off, CH)])
        return carry

    lax.fori_loop(0, NCH, step, 0)


_gather = functools.partial(
    pl.kernel,
    out_type=(jax.ShapeDtypeStruct((EE, DD), _f32),
              jax.ShapeDtypeStruct((EE,), _f32),
              jax.ShapeDtypeStruct((EE,), _f32),
              jax.ShapeDtypeStruct((EE,), _f32)),
    mesh=_mesh,
    compiler_params=_sc_params,
    scratch_types=[
        pltpu.VMEM((CH,), jnp.int32),
        pltpu.VMEM((CH,), jnp.int32),
        pltpu.VMEM((CH, DD), _f32),
        pltpu.VMEM((CH, DD), _f32),
        pltpu.VMEM((NN,), _f32),
        pltpu.VMEM((NN,), _f32),
        pltpu.VMEM((NN,), _f32),
        pltpu.VMEM((CH,), _f32),
        pltpu.VMEM((CH,), _f32),
        pltpu.VMEM((CH,), _f32),
        pltpu.SemaphoreType.DMA,
        pltpu.SemaphoreType.DMA,
    ],
)(_gather_body)


def _pre_body(nf_ref, w1a_ref, w1b_ref, be1_ref, p1_ref, p2_ref):
    nf = nf_ref[...]
    p1_ref[...] = (jnp.dot(nf, w1a_ref[...], preferred_element_type=_f32)
                   + be1_ref[...])
    p2_ref[...] = jnp.dot(nf, w1b_ref[...], preferred_element_type=_f32)


def _pre(nfp, w1a, w1b, be1):
    return pl.pallas_call(
        _pre_body,
        out_shape=[
            jax.ShapeDtypeStruct((NP, DD), _f32),
            jax.ShapeDtypeStruct((NP, DD), _f32),
        ],
    )(nfp, w1a, w1b, be1)


def _scat_ef_body(ef_hbm, dst_hbm, z128_hbm, pef_hbm, acc128, idxb, efb):
    c = lax.axis_index("c")
    s = lax.axis_index("s")
    wid = s * NC + c
    base = wid * EPT
    r0 = s * RPT

    pltpu.sync_copy(z128_hbm.at[pl.ds(r0, RPT)], acc128.at[pl.ds(r0, RPT)])
    plsc.subcore_barrier()

    def step(i, carry):
        off = base + i * CH
        pltpu.sync_copy(dst_hbm.at[pl.ds(off, CH)], idxb)
        pltpu.sync_copy(ef_hbm.at[pl.ds(off, CH)], efb)
        pltpu.sync_copy(efb, acc128.at[idxb], add=True)
        return carry

    lax.fori_loop(0, NCH, step, 0)
    plsc.subcore_barrier()

    pltpu.sync_copy(acc128.at[pl.ds(r0, RPT)], pef_hbm.at[c].at[pl.ds(r0, RPT)])


_scat_ef = functools.partial(
    pl.kernel,
    out_type=jax.ShapeDtypeStruct((NC, NP, DD), _f32),
    mesh=_mesh,
    compiler_params=_sc_params,
    scratch_types=[
        pltpu.VMEM_SHARED((NP, DD), _f32),
        pltpu.VMEM((CH,), jnp.int32),
        pltpu.VMEM((CH, DD), _f32),
    ],
)(_scat_ef_body)


def _scat_aux_body(dst_hbm, wx_hbm, wy_hbm, wz_hbm, z4_hbm, paux_hbm,
                   acc4, idxb, wxb, wyb, wzb):
    c = lax.axis_index("c")
    s = lax.axis_index("s")
    wid = s * NC + c
    base = wid * EPT

    pltpu.sync_copy(z4_hbm, acc4)
    ones16 = jnp.ones((16,), _f32)

    def step(i, carry):
        off = base + i * CHA
        pltpu.sync_copy(dst_hbm.at[pl.ds(off, CHA)], idxb)
        pltpu.sync_copy(wx_hbm.at[pl.ds(off, CHA)], wxb)
        pltpu.sync_copy(wy_hbm.at[pl.ds(off, CHA)], wyb)
        pltpu.sync_copy(wz_hbm.at[pl.ds(off, CHA)], wzb)
        for g in range(GA):
            dv = idxb[pl.ds(g * 16, 16)]
            plsc.addupdate_scatter(acc4, [dv], wxb[pl.ds(g * 16, 16)])
            plsc.addupdate_scatter(acc4, [dv + NP], wyb[pl.ds(g * 16, 16)])
            plsc.addupdate_scatter(acc4, [dv + 2 * NP], wzb[pl.ds(g * 16, 16)])
            plsc.addupdate_scatter(acc4, [dv + 3 * NP], ones16)
        return carry

    lax.fori_loop(0, NCHA, step, 0)
    pltpu.sync_copy(acc4, paux_hbm.at[wid])


_scat_aux = functools.partial(
    pl.kernel,
    out_type=jax.ShapeDtypeStruct((NW, 4 * NP), _f32),
    mesh=_mesh,
    compiler_params=_sc_params,
    scratch_types=[
        pltpu.VMEM((4 * NP,), _f32),
        pltpu.VMEM((CHA,), jnp.int32),
        pltpu.VMEM((CHA,), _f32),
        pltpu.VMEM((CHA,), _f32),
        pltpu.VMEM((CHA,), _f32),
    ],
)(_scat_aux_body)


BE = 512


def _edge_body(hp_ref, dx_ref, dy_ref, dz_ref,
               w1c_ref,
               w2_ref, be2_ref, wc1_ref, bc1_ref, wc2_ref,
               ef_ref, wx_ref, wy_ref, wz_ref):
    dx = dx_ref[...]
    dy = dy_ref[...]
    dz = dz_ref[...]
    radial = (dx * dx + dy * dy + dz * dz).reshape(BE, 1)
    h1 = jnp.maximum(hp_ref[...] + radial * w1c_ref[...], 0.0)
    ef = jnp.maximum(jnp.dot(h1, w2_ref[...], preferred_element_type=_f32)
                     + be2_ref[...], 0.0)
    c1 = jnp.maximum(jnp.dot(ef, wc1_ref[...], preferred_element_type=_f32)
                     + bc1_ref[...], 0.0)
    w = jnp.sum(c1 * wc2_ref[...], axis=1)
    ef_ref[...] = ef
    wx_ref[...] = dx * w
    wy_ref[...] = dy * w
    wz_ref[...] = dz * w


def _edge_mlp(hp, dx, dy, dz, w1c, w2, be2, wc1, bc1, wc2):
    nblk = EE // BE
    full128 = pl.BlockSpec((DD, DD), lambda i: (0, 0))
    row128 = pl.BlockSpec((1, DD), lambda i: (0, 0))
    vec = pl.BlockSpec((BE,), lambda i: (i,))
    return pl.pallas_call(
        _edge_body,
        grid=(nblk,),
        in_specs=[
            pl.BlockSpec((BE, DD), lambda i: (i, 0)),
            vec, vec, vec,
            row128,
            full128, row128, full128, row128, row128,
        ],
        out_specs=[
            pl.BlockSpec((BE, DD), lambda i: (i, 0)),
            vec, vec, vec,
        ],
        out_shape=[
            jax.ShapeDtypeStruct((EE, DD), _f32),
            jax.ShapeDtypeStruct((EE,), _f32),
            jax.ShapeDtypeStruct((EE,), _f32),
            jax.ShapeDtypeStruct((EE,), _f32),
        ],
    )(hp, dx, dy, dz, w1c, w2, be2, wc1, bc1, wc2)


def _node_body(nf_ref, ct_ref, pef_ref, paux_ref,
               wn1a_ref, wn1b_ref, bn1_ref, wn2_ref, bn2_ref,
               nfo_ref, cot_ref):
    agg = pef_ref[0] + pef_ref[1]
    pa = paux_ref[...]
    wx = jnp.sum(pa[:, 0:NP], axis=0, keepdims=True)
    wy = jnp.sum(pa[:, NP:2 * NP], axis=0, keepdims=True)
    wz = jnp.sum(pa[:, 2 * NP:3 * NP], axis=0, keepdims=True)
    cnt = jnp.maximum(jnp.sum(pa[:, 3 * NP:4 * NP], axis=0, keepdims=True), 1.0)
    inv = 1.0 / cnt
    delta = jnp.concatenate(
        [wx * inv, wy * inv, wz * inv, jnp.zeros((5, NP), _f32)], axis=0)
    cot_ref[...] = ct_ref[...] + delta
    nf = nf_ref[...]
    h = jnp.maximum(jnp.dot(nf, wn1a_ref[...], preferred_element_type=_f32)
                    + jnp.dot(agg, wn1b_ref[...], preferred_element_type=_f32)
                    + bn1_ref[...], 0.0)
    nfo_ref[...] = (nf + jnp.dot(h, wn2_ref[...], preferred_element_type=_f32)
                    + bn2_ref[...])


def _node_mlp(nf, ct8, pef, paux, wn1a, wn1b, bn1, wn2, bn2):
    return pl.pallas_call(
        _node_body,
        out_shape=[
            jax.ShapeDtypeStruct((NP, DD), _f32),
            jax.ShapeDtypeStruct((8, NP), _f32),
        ],
    )(nf, ct8, pef, paux, wn1a, wn1b, bn1, wn2, bn2)


def kernel(node_feat, coord, edge_list, We1, be1, We2, be2,
           Wn1, bn1, Wn2, bn2, Wc1, bc1, Wc2):
    src = edge_list[:, 0]
    dst = edge_list[:, 1]
    cx = coord[:, 0]
    cy = coord[:, 1]
    cz = coord[:, 2]
    ct8 = jnp.pad(coord.T, ((0, 5), (0, NP - NN)))
    nfp = jnp.pad(node_feat, ((0, NP - NN), (0, 0)))

    w1a = We1[:DD]
    w1b = We1[DD:2 * DD]
    w1c = We1[2 * DD:2 * DD + 1]
    p1, p2 = _pre(nfp, w1a, w1b, be1.reshape(1, DD))

    hp, dx, dy, dz = _gather(p1, p2, cx, cy, cz, src, dst)

    ef, wx, wy, wz = _edge_mlp(hp, dx, dy, dz, w1c,
                               We2, be2.reshape(1, DD),
                               Wc1, bc1.reshape(1, DD), Wc2.reshape(1, DD))

    z128 = jnp.zeros((NP, DD), _f32)
    z4 = jnp.zeros((4 * NP,), _f32)
    pef = _scat_ef(ef, dst, z128)
    paux = _scat_aux(dst, wx, wy, wz, z4)

    nfo, cot = _node_mlp(nfp, ct8, pef, paux,
                         Wn1[:DD], Wn1[DD:2 * DD], bn1.reshape(1, DD),
                         Wn2, bn2.reshape(1, DD))
    return (nfo[:NN], cot[:3, :NN].T)


# speedup vs baseline: 4.7318x; 1.2374x over previous
import functools

import jax
import jax.numpy as jnp
from jax import lax
from jax.experimental import pallas as pl
from jax.experimental.pallas import tpu as pltpu
from jax.experimental.pallas import tpu_sc as plsc

NN = 10000
EE = 320000
DD = 128

NC = 2
NS = 16
NW = NC * NS
EPT = EE // NW
CH = 80
NCH = EPT // CH
G = CH // 16
CHA = 400
NCHA = EPT // CHA
GA = CHA // 16
NP = 10240
RPT = NP // NS

_f32 = jnp.float32
_mesh = plsc.VectorSubcoreMesh(core_axis_name="c", subcore_axis_name="s",
                               num_cores=NC, num_subcores=NS)
_sc_params = pltpu.CompilerParams(needs_layout_passes=False)


def _gather_body(p1_hbm, p2_hbm, cx_hbm, cy_hbm, cz_hbm, src_hbm, dst_hbm,
                 hp_hbm, dx_hbm, dy_hbm, dz_hbm,
                 idx_sv, idx_dv, bs0, bd0, bs1, bd1, cxv, cyv, czv,
                 dxB, dyB, dzB, sem_s0, sem_d0, sem_s1, sem_d1):
    c = lax.axis_index("c")
    s = lax.axis_index("s")
    wid = s * NC + c
    base = wid * EPT

    pltpu.sync_copy(cx_hbm, cxv)
    pltpu.sync_copy(cy_hbm, cyv)
    pltpu.sync_copy(cz_hbm, czv)
    pltpu.sync_copy(src_hbm.at[pl.ds(base, EPT)], idx_sv)
    pltpu.sync_copy(dst_hbm.at[pl.ds(base, EPT)], idx_dv)

    def issue(ch, bs, bd, ss, sd):
        loff = ch * CH
        pltpu.async_copy(p1_hbm.at[idx_sv.at[pl.ds(loff, CH)]], bs, ss)
        pltpu.async_copy(p2_hbm.at[idx_dv.at[pl.ds(loff, CH)]], bd, sd)

    def wait(bs, bd, ss, sd):
        pltpu.make_async_copy(p1_hbm.at[idx_sv.at[pl.ds(0, CH)]], bs, ss).wait()
        pltpu.make_async_copy(p2_hbm.at[idx_dv.at[pl.ds(0, CH)]], bd, sd).wait()

    def process(ch, bs, bd):
        loff = ch * CH
        for g in range(G):
            sl16 = pl.ds(loff + g * 16, 16)
            sv = idx_sv[sl16]
            dv = idx_dv[sl16]
            dxB[sl16] = plsc.load_gather(cxv, [dv]) - plsc.load_gather(cxv, [sv])
            dyB[sl16] = plsc.load_gather(cyv, [dv]) - plsc.load_gather(cyv, [sv])
            dzB[sl16] = plsc.load_gather(czv, [dv]) - plsc.load_gather(czv, [sv])

        def row(j, carry2):
            for g in range(DD // 16):
                sl = pl.ds(g * 16, 16)
                bs[j, sl] = bs[j, sl] + bd[j, sl]
            return carry2

        lax.fori_loop(0, CH, row, 0)
        pltpu.sync_copy(bs, hp_hbm.at[pl.ds(base + loff, CH)])

    issue(0, bs0, bd0, sem_s0, sem_d0)

    def step(i, carry):
        a = 2 * i
        issue(a + 1, bs1, bd1, sem_s1, sem_d1)
        wait(bs0, bd0, sem_s0, sem_d0)
        process(a, bs0, bd0)
        issue(a + 2, bs0, bd0, sem_s0, sem_d0)
        wait(bs1, bd1, sem_s1, sem_d1)
        process(a + 1, bs1, bd1)
        return carry

    lax.fori_loop(0, (NCH - 1) // 2, step, 0)
    wait(bs0, bd0, sem_s0, sem_d0)
    process(NCH - 1, bs0, bd0)

    pltpu.sync_copy(dxB, dx_hbm.at[pl.ds(base, EPT)])
    pltpu.sync_copy(dyB, dy_hbm.at[pl.ds(base, EPT)])
    pltpu.sync_copy(dzB, dz_hbm.at[pl.ds(base, EPT)])


_gather = functools.partial(
    pl.kernel,
    out_type=(jax.ShapeDtypeStruct((EE, DD), _f32),
              jax.ShapeDtypeStruct((EE,), _f32),
              jax.ShapeDtypeStruct((EE,), _f32),
              jax.ShapeDtypeStruct((EE,), _f32)),
    mesh=_mesh,
    compiler_params=_sc_params,
    scratch_types=[
        pltpu.VMEM((EPT,), jnp.int32),
        pltpu.VMEM((EPT,), jnp.int32),
        pltpu.VMEM((CH, DD), _f32),
        pltpu.VMEM((CH, DD), _f32),
        pltpu.VMEM((CH, DD), _f32),
        pltpu.VMEM((CH, DD), _f32),
        pltpu.VMEM((NN,), _f32),
        pltpu.VMEM((NN,), _f32),
        pltpu.VMEM((NN,), _f32),
        pltpu.VMEM((EPT,), _f32),
        pltpu.VMEM((EPT,), _f32),
        pltpu.VMEM((EPT,), _f32),
        pltpu.SemaphoreType.DMA,
        pltpu.SemaphoreType.DMA,
        pltpu.SemaphoreType.DMA,
        pltpu.SemaphoreType.DMA,
    ],
)(_gather_body)


def _pre_body(nf_ref, w1a_ref, w1b_ref, be1_ref, p1_ref, p2_ref):
    nf = nf_ref[...]
    p1_ref[...] = (jnp.dot(nf, w1a_ref[...], preferred_element_type=_f32)
                   + be1_ref[...])
    p2_ref[...] = jnp.dot(nf, w1b_ref[...], preferred_element_type=_f32)


def _pre(nfp, w1a, w1b, be1):
    return pl.pallas_call(
        _pre_body,
        out_shape=[
            jax.ShapeDtypeStruct((NP, DD), _f32),
            jax.ShapeDtypeStruct((NP, DD), _f32),
        ],
    )(nfp, w1a, w1b, be1)


def _scat_ef_body(ef_hbm, dst_hbm, z128_hbm, pef_hbm, acc128, idxb, efb):
    c = lax.axis_index("c")
    s = lax.axis_index("s")
    wid = s * NC + c
    base = wid * EPT
    r0 = s * RPT

    pltpu.sync_copy(z128_hbm.at[pl.ds(r0, RPT)], acc128.at[pl.ds(r0, RPT)])
    plsc.subcore_barrier()

    def step(i, carry):
        off = base + i * CH
        pltpu.sync_copy(dst_hbm.at[pl.ds(off, CH)], idxb)
        pltpu.sync_copy(ef_hbm.at[pl.ds(off, CH)], efb)
        pltpu.sync_copy(efb, acc128.at[idxb], add=True)
        return carry

    lax.fori_loop(0, NCH, step, 0)
    plsc.subcore_barrier()

    pltpu.sync_copy(acc128.at[pl.ds(r0, RPT)], pef_hbm.at[c].at[pl.ds(r0, RPT)])


_scat_ef = functools.partial(
    pl.kernel,
    out_type=jax.ShapeDtypeStruct((NC, NP, DD), _f32),
    mesh=_mesh,
    compiler_params=_sc_params,
    scratch_types=[
        pltpu.VMEM_SHARED((NP, DD), _f32),
        pltpu.VMEM((CH,), jnp.int32),
        pltpu.VMEM((CH, DD), _f32),
    ],
)(_scat_ef_body)


def _scat_aux_body(dst_hbm, wx_hbm, wy_hbm, wz_hbm, z4_hbm, paux_hbm,
                   acc4, idxb, wxb, wyb, wzb):
    c = lax.axis_index("c")
    s = lax.axis_index("s")
    wid = s * NC + c
    base = wid * EPT

    pltpu.sync_copy(z4_hbm, acc4)
    ones16 = jnp.ones((16,), _f32)

    def step(i, carry):
        off = base + i * CHA
        pltpu.sync_copy(dst_hbm.at[pl.ds(off, CHA)], idxb)
        pltpu.sync_copy(wx_hbm.at[pl.ds(off, CHA)], wxb)
        pltpu.sync_copy(wy_hbm.at[pl.ds(off, CHA)], wyb)
        pltpu.sync_copy(wz_hbm.at[pl.ds(off, CHA)], wzb)
        for g in range(GA):
            dv = idxb[pl.ds(g * 16, 16)]
            plsc.addupdate_scatter(acc4, [dv], wxb[pl.ds(g * 16, 16)])
            plsc.addupdate_scatter(acc4, [dv + NP], wyb[pl.ds(g * 16, 16)])
            plsc.addupdate_scatter(acc4, [dv + 2 * NP], wzb[pl.ds(g * 16, 16)])
            plsc.addupdate_scatter(acc4, [dv + 3 * NP], ones16)
        return carry

    lax.fori_loop(0, NCHA, step, 0)
    pltpu.sync_copy(acc4, paux_hbm.at[wid])


_scat_aux = functools.partial(
    pl.kernel,
    out_type=jax.ShapeDtypeStruct((NW, 4 * NP), _f32),
    mesh=_mesh,
    compiler_params=_sc_params,
    scratch_types=[
        pltpu.VMEM((4 * NP,), _f32),
        pltpu.VMEM((CHA,), jnp.int32),
        pltpu.VMEM((CHA,), _f32),
        pltpu.VMEM((CHA,), _f32),
        pltpu.VMEM((CHA,), _f32),
    ],
)(_scat_aux_body)


BE = 512


def _edge_body(hp_ref, dx_ref, dy_ref, dz_ref,
               w1c_ref,
               w2_ref, be2_ref, wc1_ref, bc1_ref, wc2_ref,
               ef_ref, wx_ref, wy_ref, wz_ref):
    dx = dx_ref[...]
    dy = dy_ref[...]
    dz = dz_ref[...]
    radial = (dx * dx + dy * dy + dz * dz).reshape(BE, 1)
    h1 = jnp.maximum(hp_ref[...] + radial * w1c_ref[...], 0.0)
    ef = jnp.maximum(jnp.dot(h1, w2_ref[...], preferred_element_type=_f32)
                     + be2_ref[...], 0.0)
    c1 = jnp.maximum(jnp.dot(ef, wc1_ref[...], preferred_element_type=_f32)
                     + bc1_ref[...], 0.0)
    w = jnp.sum(c1 * wc2_ref[...], axis=1)
    ef_ref[...] = ef
    wx_ref[...] = dx * w
    wy_ref[...] = dy * w
    wz_ref[...] = dz * w


def _edge_mlp(hp, dx, dy, dz, w1c, w2, be2, wc1, bc1, wc2):
    nblk = EE // BE
    full128 = pl.BlockSpec((DD, DD), lambda i: (0, 0))
    row128 = pl.BlockSpec((1, DD), lambda i: (0, 0))
    vec = pl.BlockSpec((BE,), lambda i: (i,))
    return pl.pallas_call(
        _edge_body,
        grid=(nblk,),
        in_specs=[
            pl.BlockSpec((BE, DD), lambda i: (i, 0)),
            vec, vec, vec,
            row128,
            full128, row128, full128, row128, row128,
        ],
        out_specs=[
            pl.BlockSpec((BE, DD), lambda i: (i, 0)),
            vec, vec, vec,
        ],
        out_shape=[
            jax.ShapeDtypeStruct((EE, DD), _f32),
            jax.ShapeDtypeStruct((EE,), _f32),
            jax.ShapeDtypeStruct((EE,), _f32),
            jax.ShapeDtypeStruct((EE,), _f32),
        ],
    )(hp, dx, dy, dz, w1c, w2, be2, wc1, bc1, wc2)


def _node_body(nf_ref, ct_ref, pef_ref, paux_ref,
               wn1a_ref, wn1b_ref, bn1_ref, wn2_ref, bn2_ref,
               nfo_ref, cot_ref):
    agg = pef_ref[0] + pef_ref[1]
    pa = paux_ref[...]
    wx = jnp.sum(pa[:, 0:NP], axis=0, keepdims=True)
    wy = jnp.sum(pa[:, NP:2 * NP], axis=0, keepdims=True)
    wz = jnp.sum(pa[:, 2 * NP:3 * NP], axis=0, keepdims=True)
    cnt = jnp.maximum(jnp.sum(pa[:, 3 * NP:4 * NP], axis=0, keepdims=True), 1.0)
    inv = 1.0 / cnt
    delta = jnp.concatenate(
        [wx * inv, wy * inv, wz * inv, jnp.zeros((5, NP), _f32)], axis=0)
    cot_ref[...] = ct_ref[...] + delta
    nf = nf_ref[...]
    h = jnp.maximum(jnp.dot(nf, wn1a_ref[...], preferred_element_type=_f32)
                    + jnp.dot(agg, wn1b_ref[...], preferred_element_type=_f32)
                    + bn1_ref[...], 0.0)
    nfo_ref[...] = (nf + jnp.dot(h, wn2_ref[...], preferred_element_type=_f32)
                    + bn2_ref[...])


def _node_mlp(nf, ct8, pef, paux, wn1a, wn1b, bn1, wn2, bn2):
    return pl.pallas_call(
        _node_body,
        out_shape=[
            jax.ShapeDtypeStruct((NP, DD), _f32),
            jax.ShapeDtypeStruct((8, NP), _f32),
        ],
    )(nf, ct8, pef, paux, wn1a, wn1b, bn1, wn2, bn2)


def kernel(node_feat, coord, edge_list, We1, be1, We2, be2,
           Wn1, bn1, Wn2, bn2, Wc1, bc1, Wc2):
    src = edge_list[:, 0]
    dst = edge_list[:, 1]
    cx = coord[:, 0]
    cy = coord[:, 1]
    cz = coord[:, 2]
    ct8 = jnp.pad(coord.T, ((0, 5), (0, NP - NN)))
    nfp = jnp.pad(node_feat, ((0, NP - NN), (0, 0)))

    w1a = We1[:DD]
    w1b = We1[DD:2 * DD]
    w1c = We1[2 * DD:2 * DD + 1]
    p1, p2 = _pre(nfp, w1a, w1b, be1.reshape(1, DD))

    hp, dx, dy, dz = _gather(p1, p2, cx, cy, cz, src, dst)

    ef, wx, wy, wz = _edge_mlp(hp, dx, dy, dz, w1c,
                               We2, be2.reshape(1, DD),
                               Wc1, bc1.reshape(1, DD), Wc2.reshape(1, DD))

    z128 = jnp.zeros((NP, DD), _f32)
    z4 = jnp.zeros((4 * NP,), _f32)
    pef = _scat_ef(ef, dst, z128)
    paux = _scat_aux(dst, wx, wy, wz, z4)

    nfo, cot = _node_mlp(nfp, ct8, pef, paux,
                         Wn1[:DD], Wn1[DD:2 * DD], bn1.reshape(1, DD),
                         Wn2, bn2.reshape(1, DD))
    return (nfo[:NN], cot[:3, :NN].T)


# speedup vs baseline: 5.3818x; 1.1374x over previous
import functools

import jax
import jax.numpy as jnp
from jax import lax
from jax.experimental import pallas as pl
from jax.experimental.pallas import tpu as pltpu
from jax.experimental.pallas import tpu_sc as plsc

NN = 10000
EE = 320000
DD = 128

NC = 2
NS = 16
NW = NC * NS
EPT = EE // NW
CH = 80
NCH = EPT // CH
G = CH // 16
CHA = 400
NCHA = EPT // CHA
GA = CHA // 16
NP = 10240
RPT = NP // NS

_f32 = jnp.float32
_mesh = plsc.VectorSubcoreMesh(core_axis_name="c", subcore_axis_name="s",
                               num_cores=NC, num_subcores=NS)
_sc_params = pltpu.CompilerParams(needs_layout_passes=False)


def _gather_body(p1_hbm, p2_hbm, cx_hbm, cy_hbm, cz_hbm, src_hbm, dst_hbm,
                 hp_hbm, dx_hbm, dy_hbm, dz_hbm,
                 idx_sv, idx_dv, bs0, bd0, bs1, bd1, cxv, cyv, czv,
                 dxB, dyB, dzB, sem_s0, sem_d0, sem_s1, sem_d1):
    c = lax.axis_index("c")
    s = lax.axis_index("s")
    wid = s * NC + c
    base = wid * EPT

    pltpu.sync_copy(cx_hbm, cxv)
    pltpu.sync_copy(cy_hbm, cyv)
    pltpu.sync_copy(cz_hbm, czv)
    pltpu.sync_copy(src_hbm.at[pl.ds(base, EPT)], idx_sv)
    pltpu.sync_copy(dst_hbm.at[pl.ds(base, EPT)], idx_dv)

    def issue(ch, bs, bd, ss, sd):
        loff = ch * CH
        pltpu.async_copy(p1_hbm.at[idx_sv.at[pl.ds(loff, CH)]], bs, ss)
        pltpu.async_copy(p2_hbm.at[idx_dv.at[pl.ds(loff, CH)]], bd, sd)

    def wait(bs, bd, ss, sd):
        pltpu.make_async_copy(p1_hbm.at[idx_sv.at[pl.ds(0, CH)]], bs, ss).wait()
        pltpu.make_async_copy(p2_hbm.at[idx_dv.at[pl.ds(0, CH)]], bd, sd).wait()

    def process(ch, bs, bd):
        loff = ch * CH
        for g in range(G):
            sl16 = pl.ds(loff + g * 16, 16)
            sv = idx_sv[sl16]
            dv = idx_dv[sl16]
            dxB[sl16] = plsc.load_gather(cxv, [dv]) - plsc.load_gather(cxv, [sv])
            dyB[sl16] = plsc.load_gather(cyv, [dv]) - plsc.load_gather(cyv, [sv])
            dzB[sl16] = plsc.load_gather(czv, [dv]) - plsc.load_gather(czv, [sv])

        def row(j, carry2):
            for g in range(DD // 16):
                sl = pl.ds(g * 16, 16)
                bs[j, sl] = bs[j, sl] + bd[j, sl]
            return carry2

        lax.fori_loop(0, CH, row, 0)
        pltpu.sync_copy(bs, hp_hbm.at[pl.ds(base + loff, CH)])

    issue(0, bs0, bd0, sem_s0, sem_d0)

    def step(i, carry):
        a = 2 * i
        issue(a + 1, bs1, bd1, sem_s1, sem_d1)
        wait(bs0, bd0, sem_s0, sem_d0)
        process(a, bs0, bd0)
        issue(a + 2, bs0, bd0, sem_s0, sem_d0)
        wait(bs1, bd1, sem_s1, sem_d1)
        process(a + 1, bs1, bd1)
        return carry

    lax.fori_loop(0, (NCH - 1) // 2, step, 0)
    wait(bs0, bd0, sem_s0, sem_d0)
    process(NCH - 1, bs0, bd0)

    pltpu.sync_copy(dxB, dx_hbm.at[pl.ds(base, EPT)])
    pltpu.sync_copy(dyB, dy_hbm.at[pl.ds(base, EPT)])
    pltpu.sync_copy(dzB, dz_hbm.at[pl.ds(base, EPT)])


_gather = functools.partial(
    pl.kernel,
    out_type=(jax.ShapeDtypeStruct((EE, DD), _f32),
              jax.ShapeDtypeStruct((EE,), _f32),
              jax.ShapeDtypeStruct((EE,), _f32),
              jax.ShapeDtypeStruct((EE,), _f32)),
    mesh=_mesh,
    compiler_params=_sc_params,
    scratch_types=[
        pltpu.VMEM((EPT,), jnp.int32),
        pltpu.VMEM((EPT,), jnp.int32),
        pltpu.VMEM((CH, DD), _f32),
        pltpu.VMEM((CH, DD), _f32),
        pltpu.VMEM((CH, DD), _f32),
        pltpu.VMEM((CH, DD), _f32),
        pltpu.VMEM((NN,), _f32),
        pltpu.VMEM((NN,), _f32),
        pltpu.VMEM((NN,), _f32),
        pltpu.VMEM((EPT,), _f32),
        pltpu.VMEM((EPT,), _f32),
        pltpu.VMEM((EPT,), _f32),
        pltpu.SemaphoreType.DMA,
        pltpu.SemaphoreType.DMA,
        pltpu.SemaphoreType.DMA,
        pltpu.SemaphoreType.DMA,
    ],
)(_gather_body)


def _pre_body(nf_ref, w1a_ref, w1b_ref, be1_ref, p1_ref, p2_ref):
    nf = nf_ref[...]
    p1_ref[...] = (jnp.dot(nf, w1a_ref[...], preferred_element_type=_f32)
                   + be1_ref[...])
    p2_ref[...] = jnp.dot(nf, w1b_ref[...], preferred_element_type=_f32)


def _pre(nfp, w1a, w1b, be1):
    return pl.pallas_call(
        _pre_body,
        out_shape=[
            jax.ShapeDtypeStruct((NP, DD), _f32),
            jax.ShapeDtypeStruct((NP, DD), _f32),
        ],
    )(nfp, w1a, w1b, be1)


def _scat_ef_body(ef_hbm, dst3_hbm, z128_hbm, pef_hbm, acc128, idxv, eb0, eb1,
                  sem0, sem1):
    c = lax.axis_index("c")
    s = lax.axis_index("s")
    wid = s * NC + c
    base = wid * EPT
    r0 = s * RPT

    pltpu.sync_copy(z128_hbm.at[pl.ds(r0, RPT)], acc128.at[pl.ds(r0, RPT)])
    pltpu.sync_copy(dst3_hbm.at[wid], idxv)
    plsc.subcore_barrier()

    def issue(ch, eb, sem):
        pltpu.async_copy(ef_hbm.at[pl.ds(base + ch * CH, CH)], eb, sem)

    def wait(eb, sem):
        pltpu.make_async_copy(ef_hbm.at[pl.ds(base, CH)], eb, sem).wait()

    issue(0, eb0, sem0)

    def step(i, carry):
        a = 2 * i
        issue(a + 1, eb1, sem1)
        wait(eb0, sem0)
        pltpu.sync_copy(eb0, acc128.at[idxv.at[a]], add=True)
        issue(a + 2, eb0, sem0)
        wait(eb1, sem1)
        pltpu.sync_copy(eb1, acc128.at[idxv.at[a + 1]], add=True)
        return carry

    lax.fori_loop(0, (NCH - 1) // 2, step, 0)
    wait(eb0, sem0)
    pltpu.sync_copy(eb0, acc128.at[idxv.at[NCH - 1]], add=True)
    plsc.subcore_barrier()

    pltpu.sync_copy(acc128.at[pl.ds(r0, RPT)], pef_hbm.at[c].at[pl.ds(r0, RPT)])


_scat_ef = functools.partial(
    pl.kernel,
    out_type=jax.ShapeDtypeStruct((NC, NP, DD), _f32),
    mesh=_mesh,
    compiler_params=_sc_params,
    scratch_types=[
        pltpu.VMEM_SHARED((NP, DD), _f32),
        pltpu.VMEM((NCH, CH), jnp.int32),
        pltpu.VMEM((CH, DD), _f32),
        pltpu.VMEM((CH, DD), _f32),
        pltpu.SemaphoreType.DMA,
        pltpu.SemaphoreType.DMA,
    ],
)(_scat_ef_body)


def _scat_aux_body(dst_hbm, wx_hbm, wy_hbm, wz_hbm, z4_hbm, paux_hbm,
                   acc4, idxb, wxb, wyb, wzb):
    c = lax.axis_index("c")
    s = lax.axis_index("s")
    wid = s * NC + c
    base = wid * EPT

    pltpu.sync_copy(z4_hbm, acc4)
    ones16 = jnp.ones((16,), _f32)

    def step(i, carry):
        off = base + i * CHA
        pltpu.sync_copy(dst_hbm.at[pl.ds(off, CHA)], idxb)
        pltpu.sync_copy(wx_hbm.at[pl.ds(off, CHA)], wxb)
        pltpu.sync_copy(wy_hbm.at[pl.ds(off, CHA)], wyb)
        pltpu.sync_copy(wz_hbm.at[pl.ds(off, CHA)], wzb)
        for g in range(GA):
            dv = idxb[pl.ds(g * 16, 16)]
            plsc.addupdate_scatter(acc4, [dv], wxb[pl.ds(g * 16, 16)])
            plsc.addupdate_scatter(acc4, [dv + NP], wyb[pl.ds(g * 16, 16)])
            plsc.addupdate_scatter(acc4, [dv + 2 * NP], wzb[pl.ds(g * 16, 16)])
            plsc.addupdate_scatter(acc4, [dv + 3 * NP], ones16)
        return carry

    lax.fori_loop(0, NCHA, step, 0)
    pltpu.sync_copy(acc4, paux_hbm.at[wid])


_scat_aux = functools.partial(
    pl.kernel,
    out_type=jax.ShapeDtypeStruct((NW, 4 * NP), _f32),
    mesh=_mesh,
    compiler_params=_sc_params,
    scratch_types=[
        pltpu.VMEM((4 * NP,), _f32),
        pltpu.VMEM((CHA,), jnp.int32),
        pltpu.VMEM((CHA,), _f32),
        pltpu.VMEM((CHA,), _f32),
        pltpu.VMEM((CHA,), _f32),
    ],
)(_scat_aux_body)


BE = 512


def _edge_body(hp_ref, dx_ref, dy_ref, dz_ref,
               w1c_ref,
               w2_ref, be2_ref, wc1_ref, bc1_ref, wc2_ref,
               ef_ref, wx_ref, wy_ref, wz_ref):
    dx = dx_ref[...]
    dy = dy_ref[...]
    dz = dz_ref[...]
    radial = (dx * dx + dy * dy + dz * dz).reshape(BE, 1)
    h1 = jnp.maximum(hp_ref[...] + radial * w1c_ref[...], 0.0)
    ef = jnp.maximum(jnp.dot(h1, w2_ref[...], preferred_element_type=_f32)
                     + be2_ref[...], 0.0)
    c1 = jnp.maximum(jnp.dot(ef, wc1_ref[...], preferred_element_type=_f32)
                     + bc1_ref[...], 0.0)
    w = jnp.sum(c1 * wc2_ref[...], axis=1)
    ef_ref[...] = ef
    wx_ref[...] = dx * w
    wy_ref[...] = dy * w
    wz_ref[...] = dz * w


def _edge_mlp(hp, dx, dy, dz, w1c, w2, be2, wc1, bc1, wc2):
    nblk = EE // BE
    full128 = pl.BlockSpec((DD, DD), lambda i: (0, 0))
    row128 = pl.BlockSpec((1, DD), lambda i: (0, 0))
    vec = pl.BlockSpec((BE,), lambda i: (i,))
    return pl.pallas_call(
        _edge_body,
        grid=(nblk,),
        in_specs=[
            pl.BlockSpec((BE, DD), lambda i: (i, 0)),
            vec, vec, vec,
            row128,
            full128, row128, full128, row128, row128,
        ],
        out_specs=[
            pl.BlockSpec((BE, DD), lambda i: (i, 0)),
            vec, vec, vec,
        ],
        out_shape=[
            jax.ShapeDtypeStruct((EE, DD), _f32),
            jax.ShapeDtypeStruct((EE,), _f32),
            jax.ShapeDtypeStruct((EE,), _f32),
            jax.ShapeDtypeStruct((EE,), _f32),
        ],
    )(hp, dx, dy, dz, w1c, w2, be2, wc1, bc1, wc2)


def _node_body(nf_ref, ct_ref, pef_ref, paux_ref,
               wn1a_ref, wn1b_ref, bn1_ref, wn2_ref, bn2_ref,
               nfo_ref, cot_ref):
    agg = pef_ref[0] + pef_ref[1]
    pa = paux_ref[...]
    wx = jnp.sum(pa[:, 0:NP], axis=0, keepdims=True)
    wy = jnp.sum(pa[:, NP:2 * NP], axis=0, keepdims=True)
    wz = jnp.sum(pa[:, 2 * NP:3 * NP], axis=0, keepdims=True)
    cnt = jnp.maximum(jnp.sum(pa[:, 3 * NP:4 * NP], axis=0, keepdims=True), 1.0)
    inv = 1.0 / cnt
    delta = jnp.concatenate(
        [wx * inv, wy * inv, wz * inv, jnp.zeros((5, NP), _f32)], axis=0)
    cot_ref[...] = ct_ref[...] + delta
    nf = nf_ref[...]
    h = jnp.maximum(jnp.dot(nf, wn1a_ref[...], preferred_element_type=_f32)
                    + jnp.dot(agg, wn1b_ref[...], preferred_element_type=_f32)
                    + bn1_ref[...], 0.0)
    nfo_ref[...] = (nf + jnp.dot(h, wn2_ref[...], preferred_element_type=_f32)
                    + bn2_ref[...])


def _node_mlp(nf, ct8, pef, paux, wn1a, wn1b, bn1, wn2, bn2):
    return pl.pallas_call(
        _node_body,
        out_shape=[
            jax.ShapeDtypeStruct((NP, DD), _f32),
            jax.ShapeDtypeStruct((8, NP), _f32),
        ],
    )(nf, ct8, pef, paux, wn1a, wn1b, bn1, wn2, bn2)


def kernel(node_feat, coord, edge_list, We1, be1, We2, be2,
           Wn1, bn1, Wn2, bn2, Wc1, bc1, Wc2):
    src = edge_list[:, 0]
    dst = edge_list[:, 1]
    cx = coord[:, 0]
    cy = coord[:, 1]
    cz = coord[:, 2]
    ct8 = jnp.pad(coord.T, ((0, 5), (0, NP - NN)))
    nfp = jnp.pad(node_feat, ((0, NP - NN), (0, 0)))

    w1a = We1[:DD]
    w1b = We1[DD:2 * DD]
    w1c = We1[2 * DD:2 * DD + 1]
    p1, p2 = _pre(nfp, w1a, w1b, be1.reshape(1, DD))

    hp, dx, dy, dz = _gather(p1, p2, cx, cy, cz, src, dst)

    ef, wx, wy, wz = _edge_mlp(hp, dx, dy, dz, w1c,
                               We2, be2.reshape(1, DD),
                               Wc1, bc1.reshape(1, DD), Wc2.reshape(1, DD))

    z128 = jnp.zeros((NP, DD), _f32)
    z4 = jnp.zeros((4 * NP,), _f32)
    pef = _scat_ef(ef, dst.reshape(NW, NCH, CH), z128)
    paux = _scat_aux(dst, wx, wy, wz, z4)

    nfo, cot = _node_mlp(nfp, ct8, pef, paux,
                         Wn1[:DD], Wn1[DD:2 * DD], bn1.reshape(1, DD),
                         Wn2, bn2.reshape(1, DD))
    return (nfo[:NN], cot[:3, :NN].T)
